# baseline (speedup 1.0000x reference)
import jax
import jax.numpy as jnp
from jax import lax
from jax.experimental import pallas as pl
from jax.experimental.pallas import tpu as pltpu
from jax.experimental.pallas import tpu_sc as plsc

BATCH = 16384
FACTORS = 64
L = 16
NC, NS = 2, 16
NW = NC * NS
BPW = BATCH // NW
CW = 128
CH = BPW // CW


def _body(user_r, item_i_r, item_j_r, uf_r, if_r, out_r,
          idx_u, idx_i, idx_j, ru, ri, rj, tbuf, out_v, sem):
    wid = lax.axis_index("s") * NC + lax.axis_index("c")

    pltpu.sync_copy(user_r.at[wid], idx_u)
    pltpu.sync_copy(item_i_r.at[wid], idx_i)
    pltpu.sync_copy(item_j_r.at[wid], idx_j)

    def chunk(j, carry):
        cu = pltpu.async_copy(uf_r.at[idx_u.at[j]], ru, sem)
        ci = pltpu.async_copy(if_r.at[idx_i.at[j]], ri, sem)
        cj = pltpu.async_copy(if_r.at[idx_j.at[j]], rj, sem)
        cu.wait()
        ci.wait()
        cj.wait()

        def group(g, carry2):
            lanes = lax.iota(jnp.int32, L)
            for k in range(L):
                item = g * L + k
                acc = jnp.zeros((L,), jnp.float32)
                for c in range(FACTORS // L):
                    u = ru[item, pl.ds(c * L, L)]
                    vi = ri[item, pl.ds(c * L, L)]
                    vj = rj[item, pl.ds(c * L, L)]
                    acc = acc + u * (vi - vj)
                plsc.store_scatter(
                    tbuf, [lanes, jnp.full((L,), k, jnp.int32)], acc)
            tot = tbuf[0, :]
            for r in range(1, L):
                tot = tot + tbuf[r, :]
            out_v[pl.ds(j * CW + g * L, L)] = tot
            return carry2

        lax.fori_loop(0, CW // L, group, 0)
        return carry

    lax.fori_loop(0, CH, chunk, 0)
    pltpu.sync_copy(out_v, out_r.at[pl.ds(wid * BPW, BPW)])


def kernel(user, item_i, item_j, user_factors, item_factors):
    user3 = user.reshape(NW, CH, CW)
    ii3 = item_i.reshape(NW, CH, CW)
    ij3 = item_j.reshape(NW, CH, CW)
    mesh = plsc.VectorSubcoreMesh(core_axis_name="c", subcore_axis_name="s")
    k = pl.kernel(
        _body,
        out_type=jax.ShapeDtypeStruct((BATCH,), jnp.float32),
        mesh=mesh,
        compiler_params=pltpu.CompilerParams(
            needs_layout_passes=False, use_tc_tiling_on_sc=False),
        scratch_types=[
            pltpu.VMEM((CH, CW), jnp.int32),
            pltpu.VMEM((CH, CW), jnp.int32),
            pltpu.VMEM((CH, CW), jnp.int32),
            pltpu.VMEM((CW, FACTORS), jnp.float32),
            pltpu.VMEM((CW, FACTORS), jnp.float32),
            pltpu.VMEM((CW, FACTORS), jnp.float32),
            pltpu.VMEM((L, L), jnp.float32),
            pltpu.VMEM((BPW,), jnp.float32),
            pltpu.SemaphoreType.DMA,
        ],
    )
    return k(user3, ii3, ij3, user_factors, item_factors)
